# baseline (speedup 1.0000x reference)
import jax
import jax.numpy as jnp
from jax import lax
from jax.experimental import pallas as pl
from jax.experimental.pallas import tpu as pltpu
from jax.experimental.pallas import tpu_sc as plsc

BATCH = 4096
SEQ = 200
DIM = 64
LANES = 16

NC = 2
NS = 16
NW = NC * NS
SUB = 128
ROWS_PER_W = BATCH * SEQ // NW
NSUB = ROWS_PER_W // SUB
NBUF = 5


def _body(idx_hbm, table_hbm, pos2_hbm, out_hbm, *scratch):
    idx_v = scratch[0]
    pos2_v = scratch[1]
    rows = scratch[2:2 + NBUF]
    gsem = scratch[2 + NBUF:2 + 2 * NBUF]
    ssem = scratch[2 + 2 * NBUF:2 + 3 * NBUF]

    c = lax.axis_index("c")
    s = lax.axis_index("s")
    wid = s * NC + c

    pltpu.sync_copy(idx_hbm.at[wid], idx_v)
    pltpu.sync_copy(pos2_hbm, pos2_v)

    for b in range(NBUF):
        pltpu.make_async_copy(table_hbm.at[idx_v.at[b]], rows[b], gsem[b]).start()

    @pl.loop(0, NSUB, step=NBUF)
    def _round(g0):
        for b in range(NBUF):
            g = g0 + b
            pltpu.make_async_copy(
                table_hbm.at[idx_v.at[g]], rows[b], gsem[b]
            ).wait()

            r0 = lax.rem(g * SUB, SEQ)

            @plsc.parallel_loop(0, SUB, unroll=8)
            def _add(j):
                for k in range(DIM // LANES):
                    v = pos2_v[r0 + j, pl.ds(k * LANES, LANES)]
                    plsc.addupdate(rows[b].at[j, pl.ds(k * LANES, LANES)], v)

            out_slot = out_hbm.at[wid * NSUB + g]
            pltpu.make_async_copy(rows[b], out_slot, ssem[b]).start()
            pltpu.make_async_copy(rows[b], out_slot, ssem[b]).wait()

            nxt = g + NBUF

            @pl.when(nxt < NSUB)
            def _():
                pltpu.make_async_copy(
                    table_hbm.at[idx_v.at[nxt]], rows[b], gsem[b]
                ).start()


_scratch = (
    [pltpu.VMEM((NSUB, SUB), jnp.int32), pltpu.VMEM((2 * SEQ, DIM), jnp.float32)]
    + [pltpu.VMEM((SUB, DIM), jnp.float32) for _ in range(NBUF)]
    + [pltpu.SemaphoreType.DMA for _ in range(2 * NBUF)]
)

_kern = pl.kernel(
    _body,
    out_type=jax.ShapeDtypeStruct((NW * NSUB, SUB, DIM), jnp.float32),
    mesh=plsc.VectorSubcoreMesh(core_axis_name="c", subcore_axis_name="s"),
    scratch_types=_scratch,
    compiler_params=pltpu.CompilerParams(use_tc_tiling_on_sc=False),
    name="token_pos_embed_sc",
)


@jax.jit
def kernel(inputs, token_table, pos_table):
    b, l = inputs.shape
    _, d = token_table.shape
    idx = inputs.astype(jnp.int32).reshape(NW, NSUB, SUB)
    pos2 = jnp.concatenate([pos_table, pos_table], axis=0)
    out = _kern(idx, token_table, pos2)
    return out.reshape(b, l, d)


# speedup vs baseline: 1.0394x; 1.0394x over previous
import jax
import jax.numpy as jnp
from jax import lax
from jax.experimental import pallas as pl
from jax.experimental.pallas import tpu as pltpu
from jax.experimental.pallas import tpu_sc as plsc

BATCH = 4096
SEQ = 200
DIM = 64
LANES = 16

NC = 2
NS = 16
NW = NC * NS
SUB = 128
ROWS_PER_W = BATCH * SEQ // NW
NSUB = ROWS_PER_W // SUB
NBUF = 4


def _body(idx_hbm, table_hbm, pos2_hbm, out_hbm, *scratch):
    idx_v = scratch[0]
    pos2_v = scratch[1]
    rin = scratch[2:2 + NBUF]
    rout = scratch[2 + NBUF:2 + 2 * NBUF]
    gsem = scratch[2 + 2 * NBUF:2 + 3 * NBUF]
    ssem = scratch[2 + 3 * NBUF:2 + 4 * NBUF]

    c = lax.axis_index("c")
    s = lax.axis_index("s")
    wid = s * NC + c

    pltpu.sync_copy(idx_hbm.at[wid], idx_v)
    pltpu.sync_copy(pos2_hbm, pos2_v)

    for b in range(NBUF):
        pltpu.make_async_copy(table_hbm.at[idx_v.at[b]], rin[b], gsem[b]).start()

    @pl.loop(0, NSUB, step=NBUF)
    def _round(g0):
        for b in range(NBUF):
            g = g0 + b
            pltpu.make_async_copy(
                table_hbm.at[idx_v.at[g]], rin[b], gsem[b]
            ).wait()

            @pl.when(g0 > 0)
            def _():
                pltpu.make_async_copy(
                    rout[b], out_hbm.at[g], ssem[b]
                ).wait()

            r0 = lax.rem(g * SUB, SEQ)

            @plsc.parallel_loop(0, SUB, unroll=8)
            def _add(j):
                for k in range(DIM // LANES):
                    sl = pl.ds(k * LANES, LANES)
                    rout[b][j, sl] = rin[b][j, sl] + pos2_v[r0 + j, sl]

            pltpu.make_async_copy(
                rout[b], out_hbm.at[wid * NSUB + g], ssem[b]
            ).start()

            nxt = g + NBUF

            @pl.when(nxt < NSUB)
            def _():
                pltpu.make_async_copy(
                    table_hbm.at[idx_v.at[nxt]], rin[b], gsem[b]
                ).start()

    for b in range(NBUF):
        pltpu.make_async_copy(
            rout[b], out_hbm.at[NSUB - NBUF + b], ssem[b]
        ).wait()


_scratch = (
    [pltpu.VMEM((NSUB, SUB), jnp.int32), pltpu.VMEM((2 * SEQ, DIM), jnp.float32)]
    + [pltpu.VMEM((SUB, DIM), jnp.float32) for _ in range(2 * NBUF)]
    + [pltpu.SemaphoreType.DMA for _ in range(2 * NBUF)]
)

_kern = pl.kernel(
    _body,
    out_type=jax.ShapeDtypeStruct((NW * NSUB, SUB, DIM), jnp.float32),
    mesh=plsc.VectorSubcoreMesh(core_axis_name="c", subcore_axis_name="s"),
    scratch_types=_scratch,
    compiler_params=pltpu.CompilerParams(use_tc_tiling_on_sc=False),
    name="token_pos_embed_sc",
)


@jax.jit
def kernel(inputs, token_table, pos_table):
    b, l = inputs.shape
    _, d = token_table.shape
    idx = inputs.astype(jnp.int32).reshape(NW, NSUB, SUB)
    pos2 = jnp.concatenate([pos_table, pos_table], axis=0)
    out = _kern(idx, token_table, pos2)
    return out.reshape(b, l, d)
